# baseline (speedup 1.0000x reference)
import functools

import jax
import jax.numpy as jnp
from jax import lax
from jax.experimental import pallas as pl
from jax.experimental.pallas import tpu as pltpu
from jax.experimental.pallas import tpu_sc as plsc

CONF_T = 0.2
IOU_T = 0.6
IOU_F = IOU_T / (1.0 + IOU_T)
MAX_DET = 300
MAX_WH = 4096.0
N = 5000
NCLS = 80
KPAD = 320
OPAD = 1824
BK = 16


def _score_box_kernel(p_ref, o_ref):
    x = p_ref[0]
    obj = x[:, 4:5]
    cls = x[:, 5:5 + NCLS]
    sall = obj * cls
    best = jnp.max(sall, axis=1, keepdims=True)
    ci = lax.broadcasted_iota(jnp.int32, sall.shape, 1)
    bcls = jnp.min(jnp.where(sall == best, ci, NCLS), axis=1, keepdims=True)
    clsf = bcls.astype(jnp.float32)
    score = jnp.where(best > CONF_T, best, 0.0)
    xc, yc, w, h = x[:, 0:1], x[:, 1:2], x[:, 2:3], x[:, 3:4]
    x1 = xc - w / 2.0
    y1 = yc - h / 2.0
    x2 = xc + w / 2.0
    y2 = yc + h / 2.0
    col = lax.broadcasted_iota(jnp.int32, (N, 8), 1)
    out = jnp.zeros((N, 8), jnp.float32)
    for c, v in enumerate([score, clsf, x1, y1, x2, y2]):
        out = jnp.where(col == c, v, out)
    o_ref[...] = out


def _nms_scan(tab_h, order_h, out_h,
              tab_v, order_v, stage, kx1, ky1, kx2, ky2, outbuf):
    cid = lax.axis_index("c")
    sid = lax.axis_index("s")

    @pl.when(jnp.logical_and(cid == 0, sid == 0))
    def _():
        pltpu.sync_copy(tab_h, tab_v)
        pltpu.sync_copy(order_h, order_v)

        lanes = lax.broadcasted_iota(jnp.int32, (16,), 0)
        sent = jnp.full((16,), 1e8, jnp.float32)
        zero16 = jnp.zeros((16,), jnp.float32)

        def init_kept(i, _):
            idx = i * 16 + lanes
            plsc.store_scatter(kx1, [idx], sent)
            plsc.store_scatter(ky1, [idx], sent)
            plsc.store_scatter(kx2, [idx], sent)
            plsc.store_scatter(ky2, [idx], sent)
            return 0
        lax.fori_loop(0, KPAD // 16, init_kept, 0)

        def init_out(i, _):
            plsc.store_scatter(outbuf, [i * 16 + lanes], zero16)
            return 0
        lax.fori_loop(0, OPAD // 16, init_out, 0)

        maxdetv = jnp.full((16,), MAX_DET, jnp.int32)

        def cond(st):
            i0, kk_s, stop, kkv = st
            return (i0 < N) & (kk_s < MAX_DET) & (stop == 0)

        def body(st):
            i0, kk_s, stop, kkv = st
            cand8 = plsc.load_gather(order_v, [i0 + lanes]) * 8
            sc16 = plsc.load_gather(tab_v, [cand8])
            bmax = jnp.max(sc16)

            def one(c, kkv):
                vci = plsc.load_gather(order_v, [jnp.full((16,), i0 + c, jnp.int32)]) * 8
                vsc = plsc.load_gather(tab_v, [vci])
                vcf = plsc.load_gather(tab_v, [vci + 1])
                vrx1 = plsc.load_gather(tab_v, [vci + 2])
                vry1 = plsc.load_gather(tab_v, [vci + 3])
                vrx2 = plsc.load_gather(tab_v, [vci + 4])
                vry2 = plsc.load_gather(tab_v, [vci + 5])
                voff = vcf * MAX_WH
                vx1 = vrx1 + voff
                vy1 = vry1 + voff
                vx2 = vrx2 + voff
                vy2 = vry2 + voff
                va1 = (vx2 - vx1) * (vy2 - vy1)
                vpa1 = va1 * IOU_F + (IOU_F * 1e-9)

                def chunk(ci, acc):
                    base = ci * 64
                    for u in range(4):
                        idx = base + u * 16 + lanes
                        bx1 = plsc.load_gather(kx1, [idx])
                        by1 = plsc.load_gather(ky1, [idx])
                        bx2 = plsc.load_gather(kx2, [idx])
                        by2 = plsc.load_gather(ky2, [idx])
                        iw = jnp.minimum(vx2, bx2) - jnp.maximum(vx1, bx1)
                        ih = jnp.minimum(vy2, by2) - jnp.maximum(vy1, by1)
                        inter = jnp.maximum(iw, 0.0) * jnp.maximum(ih, 0.0)
                        va2 = (bx2 - bx1) * (by2 - by1)
                        acc = jnp.maximum(acc, inter - va2 * IOU_F)
                    return acc

                nc = (kk_s + (c + 63)) // 64
                acc = lax.fori_loop(0, nc, chunk,
                                    jnp.full((16,), -1e30, jnp.float32))
                cnt = plsc.all_reduce_population_count(acc > vpa1)
                keepv = (cnt == 0) & (vsc > 0.0) & (kkv < maxdetv)

                mk = (lanes == 0) & keepv
                plsc.store_scatter(kx1, [kkv], vx1, mask=mk)
                plsc.store_scatter(ky1, [kkv], vy1, mask=mk)
                plsc.store_scatter(kx2, [kkv], vx2, mask=mk)
                plsc.store_scatter(ky2, [kkv], vy2, mask=mk)

                row = vrx1
                row = jnp.where(lanes == 1, vry1, row)
                row = jnp.where(lanes == 2, vrx2, row)
                row = jnp.where(lanes == 3, vry2, row)
                row = jnp.where(lanes == 4, vsc, row)
                row = jnp.where(lanes == 5, vcf, row)
                plsc.store_scatter(outbuf, [kkv * 6 + lanes], row,
                                   mask=(lanes < 6) & keepv)
                return kkv + keepv.astype(jnp.int32)

            for c in range(BK):
                kkv = one(c, kkv)

            kk_s2 = jnp.max(kkv)
            stop2 = (bmax <= 0.0).astype(jnp.int32)
            return (i0 + BK, kk_s2, stop2, kkv)

        lax.while_loop(cond, body,
                       (jnp.int32(0), jnp.int32(0), jnp.int32(0),
                        jnp.zeros((16,), jnp.int32)))
        pltpu.sync_copy(outbuf.at[pl.ds(0, MAX_DET * 6)], out_h)


def kernel(preds, anchors, image_size):
    del anchors, image_size

    tab = pl.pallas_call(
        _score_box_kernel,
        out_shape=jax.ShapeDtypeStruct((N, 8), jnp.float32),
    )(preds)

    order = jnp.argsort(tab[:, 0], descending=True, stable=True).astype(jnp.int32)

    scan = functools.partial(
        pl.kernel,
        mesh=plsc.VectorSubcoreMesh(core_axis_name="c", subcore_axis_name="s"),
        out_type=jax.ShapeDtypeStruct((MAX_DET * 6,), jnp.float32),
        compiler_params=pltpu.CompilerParams(needs_layout_passes=False),
        scratch_types=[
            pltpu.VMEM((N * 8,), jnp.float32),
            pltpu.VMEM((N,), jnp.int32),
            pltpu.VMEM((16,), jnp.int32),
            pltpu.VMEM((KPAD,), jnp.float32),
            pltpu.VMEM((KPAD,), jnp.float32),
            pltpu.VMEM((KPAD,), jnp.float32),
            pltpu.VMEM((KPAD,), jnp.float32),
            pltpu.VMEM((OPAD,), jnp.float32),
        ],
    )(_nms_scan)

    det = scan(tab.reshape(N * 8), order)
    return det.reshape(1, MAX_DET, 6)


# speedup vs baseline: 1.1273x; 1.1273x over previous
import functools

import jax
import jax.numpy as jnp
from jax import lax
from jax.experimental import pallas as pl
from jax.experimental.pallas import tpu as pltpu
from jax.experimental.pallas import tpu_sc as plsc

CONF_T = 0.2
IOU_T = 0.6
IOU_F = IOU_T / (1.0 + IOU_T)
MAX_DET = 300
MAX_WH = 4096.0
N = 5000
NCLS = 80
TW = 16
KPAD = 320
OPAD = 1824
BK = 16

C_SC, C_CF, C_RX1, C_RY1, C_RX2, C_RY2 = 0, 1, 2, 3, 4, 5
C_OX1, C_OY1, C_OX2, C_OY2, C_PA, C_PAE = 6, 7, 8, 9, 10, 11


def _score_box_kernel(p_ref, o_ref):
    x = p_ref[0]
    obj = x[:, 4:5]
    cls = x[:, 5:5 + NCLS]
    sall = obj * cls
    best = jnp.max(sall, axis=1, keepdims=True)
    ci = lax.broadcasted_iota(jnp.int32, sall.shape, 1)
    bcls = jnp.min(jnp.where(sall == best, ci, NCLS), axis=1, keepdims=True)
    clsf = bcls.astype(jnp.float32)
    score = jnp.where(best > CONF_T, best, 0.0)
    xc, yc, w, h = x[:, 0:1], x[:, 1:2], x[:, 2:3], x[:, 3:4]
    rx1 = xc - w / 2.0
    ry1 = yc - h / 2.0
    rx2 = xc + w / 2.0
    ry2 = yc + h / 2.0
    off = clsf * MAX_WH
    ox1 = rx1 + off
    oy1 = ry1 + off
    ox2 = rx2 + off
    oy2 = ry2 + off
    pa = ((ox2 - ox1) * (oy2 - oy1)) * IOU_F
    pae = pa + (IOU_F * 1e-9)
    col = lax.broadcasted_iota(jnp.int32, (N, TW), 1)
    out = jnp.zeros((N, TW), jnp.float32)
    for c, v in enumerate([score, clsf, rx1, ry1, rx2, ry2,
                           ox1, oy1, ox2, oy2, pa, pae]):
        out = jnp.where(col == c, v, out)
    o_ref[...] = out


def _nms_scan(tab_h, order_h, out_h,
              tab_v, order_v, kx1, ky1, kx2, ky2, kpa, kidx, outbuf):
    cid = lax.axis_index("c")
    sid = lax.axis_index("s")

    @pl.when(jnp.logical_and(cid == 0, sid == 0))
    def _():
        pltpu.sync_copy(tab_h, tab_v)
        pltpu.sync_copy(order_h, order_v)

        lanes = lax.broadcasted_iota(jnp.int32, (16,), 0)
        sent = jnp.full((16,), 1e8, jnp.float32)
        zero16 = jnp.zeros((16,), jnp.float32)
        zero16i = jnp.zeros((16,), jnp.int32)

        def init_kept(i, _):
            idx = i * 16 + lanes
            plsc.store_scatter(kx1, [idx], sent)
            plsc.store_scatter(ky1, [idx], sent)
            plsc.store_scatter(kx2, [idx], sent)
            plsc.store_scatter(ky2, [idx], sent)
            plsc.store_scatter(kpa, [idx], zero16)
            plsc.store_scatter(kidx, [idx], zero16i)
            return 0
        lax.fori_loop(0, KPAD // 16, init_kept, 0)

        def init_out(i, _):
            plsc.store_scatter(outbuf, [i * 16 + lanes], zero16)
            return 0
        lax.fori_loop(0, OPAD // 16, init_out, 0)

        maxdetv = jnp.full((16,), MAX_DET, jnp.int32)
        neg = jnp.full((16,), -1e30, jnp.float32)

        def cond(st):
            i0, kk_s, stop = st
            return (i0 < N) & (kk_s < MAX_DET) & (stop == 0)

        def body(st):
            i0, kk_s, stop = st
            cand16 = plsc.load_gather(order_v, [i0 + lanes]) * TW
            sc16 = plsc.load_gather(tab_v, [cand16 + C_SC])
            bx1 = plsc.load_gather(tab_v, [cand16 + C_OX1])
            by1 = plsc.load_gather(tab_v, [cand16 + C_OY1])
            bx2 = plsc.load_gather(tab_v, [cand16 + C_OX2])
            by2 = plsc.load_gather(tab_v, [cand16 + C_OY2])
            bpa = plsc.load_gather(tab_v, [cand16 + C_PA])
            bmax = jnp.max(sc16)
            kksv = jnp.full((16,), kk_s, jnp.int32)
            nc = (kk_s + 63) // 64

            def splats(c):
                vci = plsc.load_gather(
                    order_v, [jnp.full((16,), i0 + c, jnp.int32)]) * TW
                vx1 = plsc.load_gather(tab_v, [vci + C_OX1])
                vy1 = plsc.load_gather(tab_v, [vci + C_OY1])
                vx2 = plsc.load_gather(tab_v, [vci + C_OX2])
                vy2 = plsc.load_gather(tab_v, [vci + C_OY2])
                vpae = plsc.load_gather(tab_v, [vci + C_PAE])
                vsc = plsc.load_gather(tab_v, [vci + C_SC])
                return vx1, vy1, vx2, vy2, vpae, vsc

            keepmask = jnp.zeros((16,), jnp.bool_)
            nblk = zero16i

            for g in range(4):
                sp = [splats(4 * g + j) for j in range(4)]

                def chunk(ci, accs, sp=sp):
                    base = ci * 64
                    accs = list(accs)
                    for u in range(4):
                        idx = base + u * 16 + lanes
                        qx1 = plsc.load_gather(kx1, [idx])
                        qy1 = plsc.load_gather(ky1, [idx])
                        qx2 = plsc.load_gather(kx2, [idx])
                        qy2 = plsc.load_gather(ky2, [idx])
                        qpa = plsc.load_gather(kpa, [idx])
                        for j in range(4):
                            vx1, vy1, vx2, vy2, _, _ = sp[j]
                            iw = jnp.minimum(vx2, qx2) - jnp.maximum(vx1, qx1)
                            ih = jnp.minimum(vy2, qy2) - jnp.maximum(vy1, qy1)
                            inter = (jnp.maximum(iw, 0.0)
                                     * jnp.maximum(ih, 0.0))
                            accs[j] = jnp.maximum(accs[j], inter - qpa)
                    return tuple(accs)

                accs = lax.fori_loop(0, nc, chunk, (neg, neg, neg, neg))

                for j in range(4):
                    c = 4 * g + j
                    vx1, vy1, vx2, vy2, vpae, vsc = sp[j]
                    memcnt = plsc.all_reduce_population_count(accs[j] > vpae)
                    iw = jnp.minimum(vx2, bx2) - jnp.maximum(vx1, bx1)
                    ih = jnp.minimum(vy2, by2) - jnp.maximum(vy1, by1)
                    inter = jnp.maximum(iw, 0.0) * jnp.maximum(ih, 0.0)
                    pairsup = (inter - bpa) > vpae
                    rel = pairsup & keepmask & (lanes < c)
                    paircnt = plsc.all_reduce_population_count(rel)
                    keep = ((memcnt == 0) & (paircnt == 0) & (vsc > 0.0)
                            & ((kksv + nblk) < maxdetv))
                    keepmask = keepmask | ((lanes == c) & keep)
                    nblk = nblk + keep.astype(jnp.int32)

            cum = plsc.cumsum(keepmask.astype(jnp.int32))
            dest = kksv + cum - 1
            plsc.store_scatter(kx1, [dest], bx1, mask=keepmask)
            plsc.store_scatter(ky1, [dest], by1, mask=keepmask)
            plsc.store_scatter(kx2, [dest], bx2, mask=keepmask)
            plsc.store_scatter(ky2, [dest], by2, mask=keepmask)
            plsc.store_scatter(kpa, [dest], bpa, mask=keepmask)
            plsc.store_scatter(kidx, [dest], cand16, mask=keepmask)

            kk_s2 = kk_s + jnp.max(cum)
            stop2 = (bmax <= 0.0).astype(jnp.int32)
            return (i0 + BK, kk_s2, stop2)

        i0f, kkf, _ = lax.while_loop(
            cond, body, (jnp.int32(0), jnp.int32(0), jnp.int32(0)))

        kkfv = zero16i + kkf

        def write_rows(i, _):
            rows = i * 16 + lanes
            msk = rows < kkfv
            vk = plsc.load_gather(kidx, [rows])
            for f, colc in enumerate([C_RX1, C_RY1, C_RX2, C_RY2, C_SC, C_CF]):
                val = plsc.load_gather(tab_v, [vk + colc])
                plsc.store_scatter(outbuf, [rows * 6 + f], val, mask=msk)
            return 0
        lax.fori_loop(0, (MAX_DET + 15) // 16, write_rows, 0)

        pltpu.sync_copy(outbuf.at[pl.ds(0, MAX_DET * 6)], out_h)


def kernel(preds, anchors, image_size):
    del anchors, image_size

    tab = pl.pallas_call(
        _score_box_kernel,
        out_shape=jax.ShapeDtypeStruct((N, TW), jnp.float32),
    )(preds)

    order = jnp.argsort(tab[:, 0], descending=True, stable=True).astype(jnp.int32)

    scan = functools.partial(
        pl.kernel,
        mesh=plsc.VectorSubcoreMesh(core_axis_name="c", subcore_axis_name="s"),
        out_type=jax.ShapeDtypeStruct((MAX_DET * 6,), jnp.float32),
        compiler_params=pltpu.CompilerParams(needs_layout_passes=False),
        scratch_types=[
            pltpu.VMEM((N * TW,), jnp.float32),
            pltpu.VMEM((N,), jnp.int32),
            pltpu.VMEM((KPAD,), jnp.float32),
            pltpu.VMEM((KPAD,), jnp.float32),
            pltpu.VMEM((KPAD,), jnp.float32),
            pltpu.VMEM((KPAD,), jnp.float32),
            pltpu.VMEM((KPAD,), jnp.float32),
            pltpu.VMEM((KPAD,), jnp.int32),
            pltpu.VMEM((OPAD,), jnp.float32),
        ],
    )(_nms_scan)

    det = scan(tab.reshape(N * TW), order)
    return det.reshape(1, MAX_DET, 6)


# speedup vs baseline: 1.1814x; 1.0480x over previous
import functools

import jax
import jax.numpy as jnp
from jax import lax
from jax.experimental import pallas as pl
from jax.experimental.pallas import tpu as pltpu
from jax.experimental.pallas import tpu_sc as plsc

CONF_T = 0.2
IOU_T = 0.6
IOU_F = IOU_T / (1.0 + IOU_T)
MAX_DET = 300
MAX_WH = 4096.0
N = 5000
NCLS = 80
TW = 16
KPAD = 320
OPAD = 1824
BK = 16

C_SC, C_CF, C_RX1, C_RY1, C_RX2, C_RY2 = 0, 1, 2, 3, 4, 5
C_OX1, C_OY1, C_OX2, C_OY2, C_PA, C_PAE = 6, 7, 8, 9, 10, 11


def _score_box_kernel(p_ref, o_ref):
    x = p_ref[0]
    obj = x[:, 4:5]
    cls = x[:, 5:5 + NCLS]
    sall = obj * cls
    best = jnp.max(sall, axis=1, keepdims=True)
    ci = lax.broadcasted_iota(jnp.int32, sall.shape, 1)
    bcls = jnp.min(jnp.where(sall == best, ci, NCLS), axis=1, keepdims=True)
    clsf = bcls.astype(jnp.float32)
    score = jnp.where(best > CONF_T, best, 0.0)
    xc, yc, w, h = x[:, 0:1], x[:, 1:2], x[:, 2:3], x[:, 3:4]
    rx1 = xc - w / 2.0
    ry1 = yc - h / 2.0
    rx2 = xc + w / 2.0
    ry2 = yc + h / 2.0
    off = clsf * MAX_WH
    ox1 = rx1 + off
    oy1 = ry1 + off
    ox2 = rx2 + off
    oy2 = ry2 + off
    pa = ((ox2 - ox1) * (oy2 - oy1)) * IOU_F
    pae = pa + (IOU_F * 1e-9)
    for c, v in enumerate([score, clsf, rx1, ry1, rx2, ry2,
                           ox1, oy1, ox2, oy2, pa, pae]):
        o_ref[:, c:c + 1] = v
    o_ref[:, 12:16] = jnp.zeros((N, 4), jnp.float32)


def _nms_scan(tab_h, order_h, out_h,
              tab_v, order_v, kx1, ky1, kx2, ky2, kpa, kidx, outbuf):
    cid = lax.axis_index("c")
    sid = lax.axis_index("s")

    @pl.when(jnp.logical_and(cid == 0, sid == 0))
    def _():
        pltpu.sync_copy(tab_h, tab_v)
        pltpu.sync_copy(order_h, order_v)

        lanes = lax.broadcasted_iota(jnp.int32, (16,), 0)
        sent = jnp.full((16,), 1e8, jnp.float32)
        zero16 = jnp.zeros((16,), jnp.float32)
        zero16i = jnp.zeros((16,), jnp.int32)

        def init_kept(i, _):
            idx = i * 16 + lanes
            plsc.store_scatter(kx1, [idx], sent)
            plsc.store_scatter(ky1, [idx], sent)
            plsc.store_scatter(kx2, [idx], sent)
            plsc.store_scatter(ky2, [idx], sent)
            plsc.store_scatter(kpa, [idx], zero16)
            plsc.store_scatter(kidx, [idx], zero16i)
            return 0
        lax.fori_loop(0, KPAD // 16, init_kept, 0)

        def init_out(i, _):
            plsc.store_scatter(outbuf, [i * 16 + lanes], zero16)
            return 0
        lax.fori_loop(0, OPAD // 16, init_out, 0)

        maxdetv = jnp.full((16,), MAX_DET, jnp.int32)
        neg = jnp.full((16,), -1e30, jnp.float32)

        def cond(st):
            i0, kk_s, stop = st
            return (i0 < N) & (kk_s < MAX_DET) & (stop == 0)

        def body(st):
            i0, kk_s, stop = st
            cand16 = plsc.load_gather(order_v, [i0 + lanes]) * TW
            sc16 = plsc.load_gather(tab_v, [cand16 + C_SC])
            bx1 = plsc.load_gather(tab_v, [cand16 + C_OX1])
            by1 = plsc.load_gather(tab_v, [cand16 + C_OY1])
            bx2 = plsc.load_gather(tab_v, [cand16 + C_OX2])
            by2 = plsc.load_gather(tab_v, [cand16 + C_OY2])
            bpa = plsc.load_gather(tab_v, [cand16 + C_PA])
            bmax = jnp.max(sc16)
            kksv = jnp.full((16,), kk_s, jnp.int32)
            nc = (kk_s + 63) // 64

            def splats(c):
                vci = plsc.load_gather(
                    order_v, [jnp.full((16,), i0 + c, jnp.int32)]) * TW
                vx1 = plsc.load_gather(tab_v, [vci + C_OX1])
                vy1 = plsc.load_gather(tab_v, [vci + C_OY1])
                vx2 = plsc.load_gather(tab_v, [vci + C_OX2])
                vy2 = plsc.load_gather(tab_v, [vci + C_OY2])
                vpae = plsc.load_gather(tab_v, [vci + C_PAE])
                vsc = plsc.load_gather(tab_v, [vci + C_SC])
                return vx1, vy1, vx2, vy2, vpae, vsc

            keepmask = jnp.zeros((16,), jnp.bool_)
            nblk = zero16i

            for g in range(4):
                sp = [splats(4 * g + j) for j in range(4)]

                def chunk(ci, accs, sp=sp):
                    base = ci * 64
                    accs = list(accs)
                    for u in range(4):
                        idx = base + u * 16 + lanes
                        qx1 = plsc.load_gather(kx1, [idx])
                        qy1 = plsc.load_gather(ky1, [idx])
                        qx2 = plsc.load_gather(kx2, [idx])
                        qy2 = plsc.load_gather(ky2, [idx])
                        qpa = plsc.load_gather(kpa, [idx])
                        for j in range(4):
                            vx1, vy1, vx2, vy2, _, _ = sp[j]
                            iw = jnp.minimum(vx2, qx2) - jnp.maximum(vx1, qx1)
                            ih = jnp.minimum(vy2, qy2) - jnp.maximum(vy1, qy1)
                            inter = (jnp.maximum(iw, 0.0)
                                     * jnp.maximum(ih, 0.0))
                            accs[j] = jnp.maximum(accs[j], inter - qpa)
                    return tuple(accs)

                accs = lax.fori_loop(0, nc, chunk, (neg, neg, neg, neg))

                for j in range(4):
                    c = 4 * g + j
                    vx1, vy1, vx2, vy2, vpae, vsc = sp[j]
                    memcnt = plsc.all_reduce_population_count(accs[j] > vpae)
                    iw = jnp.minimum(vx2, bx2) - jnp.maximum(vx1, bx1)
                    ih = jnp.minimum(vy2, by2) - jnp.maximum(vy1, by1)
                    inter = jnp.maximum(iw, 0.0) * jnp.maximum(ih, 0.0)
                    pairsup = (inter - bpa) > vpae
                    rel = pairsup & keepmask & (lanes < c)
                    paircnt = plsc.all_reduce_population_count(rel)
                    keep = ((memcnt == 0) & (paircnt == 0) & (vsc > 0.0)
                            & ((kksv + nblk) < maxdetv))
                    keepmask = keepmask | ((lanes == c) & keep)
                    nblk = nblk + keep.astype(jnp.int32)

            cum = plsc.cumsum(keepmask.astype(jnp.int32))
            dest = kksv + cum - 1
            plsc.store_scatter(kx1, [dest], bx1, mask=keepmask)
            plsc.store_scatter(ky1, [dest], by1, mask=keepmask)
            plsc.store_scatter(kx2, [dest], bx2, mask=keepmask)
            plsc.store_scatter(ky2, [dest], by2, mask=keepmask)
            plsc.store_scatter(kpa, [dest], bpa, mask=keepmask)
            plsc.store_scatter(kidx, [dest], cand16, mask=keepmask)

            kk_s2 = kk_s + jnp.max(cum)
            stop2 = (bmax <= 0.0).astype(jnp.int32)
            return (i0 + BK, kk_s2, stop2)

        i0f, kkf, _ = lax.while_loop(
            cond, body, (jnp.int32(0), jnp.int32(0), jnp.int32(0)))

        kkfv = zero16i + kkf

        def write_rows(i, _):
            rows = i * 16 + lanes
            msk = rows < kkfv
            vk = plsc.load_gather(kidx, [rows])
            for f, colc in enumerate([C_RX1, C_RY1, C_RX2, C_RY2, C_SC, C_CF]):
                val = plsc.load_gather(tab_v, [vk + colc])
                plsc.store_scatter(outbuf, [rows * 6 + f], val, mask=msk)
            return 0
        lax.fori_loop(0, (MAX_DET + 15) // 16, write_rows, 0)

        pltpu.sync_copy(outbuf.at[pl.ds(0, MAX_DET * 6)], out_h)


def kernel(preds, anchors, image_size):
    del anchors, image_size

    tab = pl.pallas_call(
        _score_box_kernel,
        out_shape=jax.ShapeDtypeStruct((N, TW), jnp.float32),
    )(preds)

    order = jnp.argsort(tab[:, 0], descending=True, stable=True).astype(jnp.int32)

    scan = functools.partial(
        pl.kernel,
        mesh=plsc.VectorSubcoreMesh(core_axis_name="c", subcore_axis_name="s"),
        out_type=jax.ShapeDtypeStruct((MAX_DET * 6,), jnp.float32),
        compiler_params=pltpu.CompilerParams(needs_layout_passes=False),
        scratch_types=[
            pltpu.VMEM((N * TW,), jnp.float32),
            pltpu.VMEM((N,), jnp.int32),
            pltpu.VMEM((KPAD,), jnp.float32),
            pltpu.VMEM((KPAD,), jnp.float32),
            pltpu.VMEM((KPAD,), jnp.float32),
            pltpu.VMEM((KPAD,), jnp.float32),
            pltpu.VMEM((KPAD,), jnp.float32),
            pltpu.VMEM((KPAD,), jnp.int32),
            pltpu.VMEM((OPAD,), jnp.float32),
        ],
    )(_nms_scan)

    det = scan(tab.reshape(N * TW), order)
    return det.reshape(1, MAX_DET, 6)
